# baseline (speedup 1.0000x reference)
import functools

import jax
import jax.numpy as jnp
from jax import lax
from jax.experimental import pallas as pl
from jax.experimental.pallas import tpu as pltpu
from jax.experimental.pallas import tpu_sc as plsc

N_TOTAL = 320000
N_SEG = 512
HID = 128
STARTS_LEN = 640

_NSLICE = 2
_SLICE_ROWS = N_TOTAL // _NSLICE
_DENSE_R = 2000
_CHUNK = 448
_NW = 32
_SEG_PER_W = N_SEG // _NW

def _weight_scale(p_ref, w1_ref, b1_ref, w2_ref, b2_ref):
    hid = jnp.maximum(
        jnp.dot(p_ref[...], w1_ref[...], preferred_element_type=jnp.float32)
        + b1_ref[...], 0.0)
    z = jnp.sum(hid * w2_ref[...], axis=1, keepdims=True) + b2_ref[0, 0]
    return 1.0 / (1.0 + jnp.exp(-z))


def _dense_hist_kernel(x_ref, p_ref, b3_ref, wp_ref, bp_ref, w1_ref, b1_ref,
                       w2_ref, b2_ref, out_ref, counts_ref, starts_ref):
    t = pl.program_id(0)
    nt = pl.num_programs(0)

    @pl.when(t == 0)
    def _init():
        counts_ref[...] = jnp.zeros_like(counts_ref)

    b = b3_ref[0, 0, :]
    bmin = jnp.min(b)
    bmax = jnp.max(b)
    for w in range(N_SEG // 64):
        lo = w * 64

        @pl.when((bmin < lo + 64) & (bmax >= lo))
        def _win(lo=lo):
            ids = lo + lax.broadcasted_iota(jnp.int32, (1, 64), 1)
            oh = (b[:, None] == ids).astype(jnp.float32)
            counts_ref[:, lo:lo + 64] += jnp.sum(oh, axis=0)[None, :]

    h = jnp.dot(x_ref[...], wp_ref[...],
                preferred_element_type=jnp.float32) + bp_ref[...]
    w = _weight_scale(p_ref, w1_ref, b1_ref, w2_ref, b2_ref)
    out_ref[...] = h * w

    @pl.when(t == nt - 1)
    def _finalize():
        cnt = counts_ref[...]
        row = lax.broadcasted_iota(jnp.int32, (N_SEG, STARTS_LEN), 0)
        col = lax.broadcasted_iota(jnp.int32, (N_SEG, STARTS_LEN), 1)
        tri = (row < col).astype(jnp.float32)
        st = jnp.dot(cnt, tri, preferred_element_type=jnp.float32,
                     precision=lax.Precision.HIGHEST)
        starts_ref[...] = st.astype(jnp.int32)


def _dense_plain_kernel(x_ref, p_ref, wp_ref, bp_ref, w1_ref, b1_ref,
                        w2_ref, b2_ref, out_ref):
    h = jnp.dot(x_ref[...], wp_ref[...],
                preferred_element_type=jnp.float32) + bp_ref[...]
    w = _weight_scale(p_ref, w1_ref, b1_ref, w2_ref, b2_ref)
    out_ref[...] = h * w


def _run_dense(x, probs, batch, Wp, bp, W1, b1, W2, b2):
    nt = _SLICE_ROWS // _DENSE_R
    rb = N_TOTAL // nt
    batch3 = batch.reshape(nt, 1, rb)
    w1p = jnp.zeros((8, HID), jnp.float32).at[:, :32].set(W1)
    b1p = jnp.zeros((1, HID), jnp.float32).at[0, :32].set(b1)
    w2p = jnp.zeros((1, HID), jnp.float32).at[0, :32].set(W2[:, 0])
    b2p = jnp.full((1, HID), b2[0], jnp.float32)
    wpp = Wp
    bpp = bp.reshape(1, HID)

    wspecs = [
        pl.BlockSpec((HID, HID), lambda i: (0, 0)),
        pl.BlockSpec((1, HID), lambda i: (0, 0)),
        pl.BlockSpec((8, HID), lambda i: (0, 0)),
        pl.BlockSpec((1, HID), lambda i: (0, 0)),
        pl.BlockSpec((1, HID), lambda i: (0, 0)),
        pl.BlockSpec((1, HID), lambda i: (0, 0)),
    ]
    wargs = (wpp, bpp, w1p, b1p, w2p, b2p)

    hw0, counts, starts = pl.pallas_call(
        _dense_hist_kernel,
        grid=(nt,),
        in_specs=[
            pl.BlockSpec((_DENSE_R, HID), lambda i: (i, 0)),
            pl.BlockSpec((_DENSE_R, 8), lambda i: (i, 0)),
            pl.BlockSpec((1, 1, rb), lambda i: (i, 0, 0)),
        ] + wspecs,
        out_specs=[pl.BlockSpec((_DENSE_R, HID), lambda i: (i, 0)),
                   pl.BlockSpec((1, N_SEG), lambda i: (0, 0)),
                   pl.BlockSpec((1, STARTS_LEN), lambda i: (0, 0))],
        out_shape=[jax.ShapeDtypeStruct((_SLICE_ROWS, HID), jnp.float32),
                   jax.ShapeDtypeStruct((1, N_SEG), jnp.float32),
                   jax.ShapeDtypeStruct((1, STARTS_LEN), jnp.int32)],
    )(x, probs, batch3, *wargs)

    hw1 = pl.pallas_call(
        _dense_plain_kernel,
        grid=(nt,),
        in_specs=[
            pl.BlockSpec((_DENSE_R, HID), lambda i: (i + nt, 0)),
            pl.BlockSpec((_DENSE_R, 8), lambda i: (i + nt, 0)),
        ] + wspecs,
        out_specs=pl.BlockSpec((_DENSE_R, HID), lambda i: (i, 0)),
        out_shape=jax.ShapeDtypeStruct((_SLICE_ROWS, HID), jnp.float32),
    )(x, probs, *wargs)

    return hw0, hw1, counts.reshape(N_SEG, 1), starts.reshape(STARTS_LEN)


def _sc_reduce_body(hw_hbm, starts_hbm, sum_hbm, max_hbm, buf_v,
                    st_v, sum_v, max_v, sem, *, lo_s, hi_s):
    c = lax.axis_index("c")
    s = lax.axis_index("s")
    wid = s * 2 + c
    seg0 = wid * _SEG_PER_W
    n_s = hi_s - lo_s

    pltpu.sync_copy(starts_hbm.at[pl.ds(seg0, 24)], st_v)

    zero = jnp.zeros((16,), jnp.float32)
    ninf = jnp.full((16,), -jnp.inf, jnp.float32)
    for k in range(_SEG_PER_W):
        for cc in range(8):
            sum_v[pl.ds(k * HID + cc * 16, 16)] = zero
            max_v[pl.ds(k * HID + cc * 16, 16)] = ninf

    sa = st_v[pl.ds(0, 16)]
    sb = st_v[pl.ds(8, 16)]

    def stv(k):
        g = sa[k] if k < 16 else sb[k - 8]
        return jnp.clip(g, lo_s, hi_s) - lo_s

    r0 = stv(0)
    r1 = stv(_SEG_PER_W)
    nch = (r1 - r0 + _CHUNK - 1) // _CHUNK

    def process(buf, rcc, off):
        for k in range(_SEG_PER_W):
            lo = jnp.clip(stv(k) - rcc, off, _CHUNK)
            hi = jnp.clip(stv(k + 1) - rcc, off, _CHUNK)

            @pl.when(hi > lo)
            def _seg(k=k, lo=lo, hi=hi):
                accs = tuple(sum_v[pl.ds(k * HID + cc * 16, 16)] for cc in range(8))
                accm = tuple(max_v[pl.ds(k * HID + cc * 16, 16)] for cc in range(8))

                def row_body(j, acc):
                    new_s = [None] * 8
                    new_m = [None] * 8
                    for cc in range(8):
                        v = buf[pl.ds(j * HID + cc * 16, 16)]
                        new_s[cc] = acc[cc] + v
                        new_m[cc] = jnp.maximum(acc[8 + cc], v)
                    return tuple(new_s) + tuple(new_m)

                res = lax.fori_loop(lo, hi, row_body, accs + accm)
                for cc in range(8):
                    sum_v[pl.ds(k * HID + cc * 16, 16)] = res[cc]
                    max_v[pl.ds(k * HID + cc * 16, 16)] = res[8 + cc]

    def chunk_body(ci, carry):
        rc = r0 + ci * _CHUNK
        rcc = jnp.minimum(rc, n_s - _CHUNK)
        pltpu.async_copy(hw_hbm.at[pl.ds(rcc * HID, _CHUNK * HID)],
                         buf_v, sem).wait()
        process(buf_v, rcc, rc - rcc)
        return carry

    lax.fori_loop(0, nch, chunk_body, 0)

    pltpu.sync_copy(sum_v, sum_hbm.at[pl.ds(seg0 * HID, _SEG_PER_W * HID)])
    pltpu.sync_copy(max_v, max_hbm.at[pl.ds(seg0 * HID, _SEG_PER_W * HID)])


def _run_sc_reduce(hw, starts, lo_s, hi_s):
    mesh = plsc.VectorSubcoreMesh(core_axis_name="c", subcore_axis_name="s")
    body = functools.partial(_sc_reduce_body, lo_s=lo_s, hi_s=hi_s)
    kern = functools.partial(
        pl.kernel,
        mesh=mesh,
        out_type=[jax.ShapeDtypeStruct((N_SEG * HID,), jnp.float32),
                  jax.ShapeDtypeStruct((N_SEG * HID,), jnp.float32)],
        scratch_types=[
            pltpu.VMEM((_CHUNK * HID,), jnp.float32),
            pltpu.VMEM((24,), jnp.int32),
            pltpu.VMEM((_SEG_PER_W * HID,), jnp.float32),
            pltpu.VMEM((_SEG_PER_W * HID,), jnp.float32),
            pltpu.SemaphoreType.DMA,
        ],
    )(body)
    sum_f, max_f = kern(hw.reshape(_SLICE_ROWS * HID), starts)
    return sum_f.reshape(N_SEG, HID), max_f.reshape(N_SEG, HID)


def _combine_kernel(s0_ref, s1_ref, m0_ref, m1_ref, cnt_ref, wt_ref, wb_ref,
                    bp_ref, out_ref):
    r = 1.0 / jnp.maximum(cnt_ref[...], 1.0)
    mean = (s0_ref[...] + s1_ref[...]) * r
    mx = jnp.maximum(m0_ref[...], m1_ref[...])
    out_ref[...] = (
        jnp.dot(mean, wt_ref[...], preferred_element_type=jnp.float32)
        + jnp.dot(mx, wb_ref[...], preferred_element_type=jnp.float32)
        + bp_ref[...])


def _run_combine(s0, s1, m0, m1, cnt_col, Wpost, bpost):
    return pl.pallas_call(
        _combine_kernel,
        in_specs=[
            pl.BlockSpec((N_SEG, HID), lambda: (0, 0)),
            pl.BlockSpec((N_SEG, HID), lambda: (0, 0)),
            pl.BlockSpec((N_SEG, HID), lambda: (0, 0)),
            pl.BlockSpec((N_SEG, HID), lambda: (0, 0)),
            pl.BlockSpec((N_SEG, 1), lambda: (0, 0)),
            pl.BlockSpec((HID, HID), lambda: (0, 0)),
            pl.BlockSpec((HID, HID), lambda: (0, 0)),
            pl.BlockSpec((1, HID), lambda: (0, 0)),
        ],
        out_specs=pl.BlockSpec((N_SEG, HID), lambda: (0, 0)),
        out_shape=jax.ShapeDtypeStruct((N_SEG, HID), jnp.float32),
    )(s0, s1, m0, m1, cnt_col, Wpost[:HID], Wpost[HID:],
      bpost.reshape(1, HID))


def kernel(node_embeddings, batch, var_property_probs, node_types,
           Wp, bp, W1, b1, W2, b2, Wpost, bpost):
    del node_types
    hw0, hw1, cnt_col, starts = _run_dense(
        node_embeddings, var_property_probs, batch, Wp, bp, W1, b1, W2, b2)
    s0, m0 = _run_sc_reduce(hw0, starts, 0, _SLICE_ROWS)
    s1, m1 = _run_sc_reduce(hw1, starts, _SLICE_ROWS, N_TOTAL)
    return _run_combine(s0, s1, m0, m1, cnt_col, Wpost, bpost)


# speedup vs baseline: 1.1783x; 1.1783x over previous
import functools

import jax
import jax.numpy as jnp
from jax import lax
from jax.experimental import pallas as pl
from jax.experimental.pallas import tpu as pltpu
from jax.experimental.pallas import tpu_sc as plsc

N_TOTAL = 320000
N_SEG = 512
HID = 128
STARTS_LEN = 640

_SLICE_ROWS = N_TOTAL
_DENSE_R = 4000
_CHUNK = 448
_NW = 32
_SEG_PER_W = N_SEG // _NW

def _weight_scale(p_ref, w1_ref, b1_ref, w2_ref, b2_ref):
    hid = jnp.maximum(
        jnp.dot(p_ref[...], w1_ref[...], preferred_element_type=jnp.float32)
        + b1_ref[...], 0.0)
    z = jnp.sum(hid * w2_ref[...], axis=1, keepdims=True) + b2_ref[0, 0]
    return 1.0 / (1.0 + jnp.exp(-z))


def _dense_hist_kernel(x_ref, p_ref, b3_ref, wp_ref, bp_ref, w1_ref, b1_ref,
                       w2_ref, b2_ref, out_ref, counts_ref, starts_ref):
    t = pl.program_id(0)
    nt = pl.num_programs(0)

    @pl.when(t == 0)
    def _init():
        counts_ref[...] = jnp.zeros_like(counts_ref)

    b = b3_ref[0, 0, :]
    bmin = jnp.min(b)
    bmax = jnp.max(b)
    for w in range(N_SEG // 64):
        lo = w * 64

        @pl.when((bmin < lo + 64) & (bmax >= lo))
        def _win(lo=lo):
            ids = lo + lax.broadcasted_iota(jnp.int32, (1, 64), 1)
            oh = (b[:, None] == ids).astype(jnp.float32)
            counts_ref[:, lo:lo + 64] += jnp.sum(oh, axis=0)[None, :]

    h = jnp.dot(x_ref[...], wp_ref[...],
                preferred_element_type=jnp.float32) + bp_ref[...]
    w = _weight_scale(p_ref, w1_ref, b1_ref, w2_ref, b2_ref)
    out_ref[...] = h * w

    @pl.when(t == nt - 1)
    def _finalize():
        cnt = counts_ref[...]
        row = lax.broadcasted_iota(jnp.int32, (N_SEG, STARTS_LEN), 0)
        col = lax.broadcasted_iota(jnp.int32, (N_SEG, STARTS_LEN), 1)
        tri = (row < col).astype(jnp.float32)
        st = jnp.dot(cnt, tri, preferred_element_type=jnp.float32,
                     precision=lax.Precision.HIGHEST)
        starts_ref[...] = st.astype(jnp.int32)


def _run_dense(x, probs, batch, Wp, bp, W1, b1, W2, b2):
    nt = _SLICE_ROWS // _DENSE_R
    rb = N_TOTAL // nt
    batch3 = batch.reshape(nt, 1, rb)
    w1p = jnp.zeros((8, HID), jnp.float32).at[:, :32].set(W1)
    b1p = jnp.zeros((1, HID), jnp.float32).at[0, :32].set(b1)
    w2p = jnp.zeros((1, HID), jnp.float32).at[0, :32].set(W2[:, 0])
    b2p = jnp.full((1, HID), b2[0], jnp.float32)
    wpp = Wp
    bpp = bp.reshape(1, HID)

    wspecs = [
        pl.BlockSpec((HID, HID), lambda i: (0, 0)),
        pl.BlockSpec((1, HID), lambda i: (0, 0)),
        pl.BlockSpec((8, HID), lambda i: (0, 0)),
        pl.BlockSpec((1, HID), lambda i: (0, 0)),
        pl.BlockSpec((1, HID), lambda i: (0, 0)),
        pl.BlockSpec((1, HID), lambda i: (0, 0)),
    ]
    wargs = (wpp, bpp, w1p, b1p, w2p, b2p)

    hw0, counts, starts = pl.pallas_call(
        _dense_hist_kernel,
        grid=(nt,),
        in_specs=[
            pl.BlockSpec((_DENSE_R, HID), lambda i: (i, 0)),
            pl.BlockSpec((_DENSE_R, 8), lambda i: (i, 0)),
            pl.BlockSpec((1, 1, rb), lambda i: (i, 0, 0)),
        ] + wspecs,
        out_specs=[pl.BlockSpec((_DENSE_R, HID), lambda i: (i, 0)),
                   pl.BlockSpec((1, N_SEG), lambda i: (0, 0)),
                   pl.BlockSpec((1, STARTS_LEN), lambda i: (0, 0))],
        out_shape=[jax.ShapeDtypeStruct((_SLICE_ROWS, HID), jnp.float32),
                   jax.ShapeDtypeStruct((1, N_SEG), jnp.float32),
                   jax.ShapeDtypeStruct((1, STARTS_LEN), jnp.int32)],
    )(x, probs, batch3, *wargs)

    return hw0, counts.reshape(N_SEG, 1), starts.reshape(STARTS_LEN)


def _sc_reduce_body(hw_hbm, starts_hbm, sum_hbm, max_hbm, buf_v,
                    st_v, sum_v, max_v, sem, *, lo_s, hi_s):
    c = lax.axis_index("c")
    s = lax.axis_index("s")
    wid = s * 2 + c
    seg0 = wid * _SEG_PER_W
    n_s = hi_s - lo_s

    pltpu.sync_copy(starts_hbm.at[pl.ds(seg0, 24)], st_v)

    zero = jnp.zeros((16,), jnp.float32)
    ninf = jnp.full((16,), -jnp.inf, jnp.float32)
    for k in range(_SEG_PER_W):
        for cc in range(8):
            sum_v[pl.ds(k * HID + cc * 16, 16)] = zero
            max_v[pl.ds(k * HID + cc * 16, 16)] = ninf

    sa = st_v[pl.ds(0, 16)]
    sb = st_v[pl.ds(8, 16)]

    def stv(k):
        g = sa[k] if k < 16 else sb[k - 8]
        return jnp.clip(g, lo_s, hi_s) - lo_s

    r0 = stv(0)
    r1 = stv(_SEG_PER_W)
    nch = (r1 - r0 + _CHUNK - 1) // _CHUNK

    def process(buf, rcc, off):
        for k in range(_SEG_PER_W):
            lo = jnp.clip(stv(k) - rcc, off, _CHUNK)
            hi = jnp.clip(stv(k + 1) - rcc, off, _CHUNK)

            @pl.when(hi > lo)
            def _seg(k=k, lo=lo, hi=hi):
                accs = tuple(sum_v[pl.ds(k * HID + cc * 16, 16)] for cc in range(8))
                accm = tuple(max_v[pl.ds(k * HID + cc * 16, 16)] for cc in range(8))

                def row_body(j, acc):
                    new_s = [None] * 8
                    new_m = [None] * 8
                    for cc in range(8):
                        v = buf[pl.ds(j * HID + cc * 16, 16)]
                        new_s[cc] = acc[cc] + v
                        new_m[cc] = jnp.maximum(acc[8 + cc], v)
                    return tuple(new_s) + tuple(new_m)

                res = lax.fori_loop(lo, hi, row_body, accs + accm)
                for cc in range(8):
                    sum_v[pl.ds(k * HID + cc * 16, 16)] = res[cc]
                    max_v[pl.ds(k * HID + cc * 16, 16)] = res[8 + cc]

    def chunk_body(ci, carry):
        rc = r0 + ci * _CHUNK
        rcc = jnp.minimum(rc, n_s - _CHUNK)
        pltpu.async_copy(hw_hbm.at[pl.ds(rcc * HID, _CHUNK * HID)],
                         buf_v, sem).wait()
        process(buf_v, rcc, rc - rcc)
        return carry

    lax.fori_loop(0, nch, chunk_body, 0)

    pltpu.sync_copy(sum_v, sum_hbm.at[pl.ds(seg0 * HID, _SEG_PER_W * HID)])
    pltpu.sync_copy(max_v, max_hbm.at[pl.ds(seg0 * HID, _SEG_PER_W * HID)])


def _run_sc_reduce(hw, starts, lo_s, hi_s):
    mesh = plsc.VectorSubcoreMesh(core_axis_name="c", subcore_axis_name="s")
    body = functools.partial(_sc_reduce_body, lo_s=lo_s, hi_s=hi_s)
    kern = functools.partial(
        pl.kernel,
        mesh=mesh,
        out_type=[jax.ShapeDtypeStruct((N_SEG * HID,), jnp.float32),
                  jax.ShapeDtypeStruct((N_SEG * HID,), jnp.float32)],
        scratch_types=[
            pltpu.VMEM((_CHUNK * HID,), jnp.float32),
            pltpu.VMEM((24,), jnp.int32),
            pltpu.VMEM((_SEG_PER_W * HID,), jnp.float32),
            pltpu.VMEM((_SEG_PER_W * HID,), jnp.float32),
            pltpu.SemaphoreType.DMA,
        ],
    )(body)
    sum_f, max_f = kern(hw.reshape(_SLICE_ROWS * HID), starts)
    return sum_f.reshape(N_SEG, HID), max_f.reshape(N_SEG, HID)


def _combine_kernel(s0_ref, m0_ref, cnt_ref, wt_ref, wb_ref,
                    bp_ref, out_ref):
    r = 1.0 / jnp.maximum(cnt_ref[...], 1.0)
    mean = s0_ref[...] * r
    mx = m0_ref[...]
    out_ref[...] = (
        jnp.dot(mean, wt_ref[...], preferred_element_type=jnp.float32)
        + jnp.dot(mx, wb_ref[...], preferred_element_type=jnp.float32)
        + bp_ref[...])


def _run_combine(s0, m0, cnt_col, Wpost, bpost):
    return pl.pallas_call(
        _combine_kernel,
        in_specs=[
            pl.BlockSpec((N_SEG, HID), lambda: (0, 0)),
            pl.BlockSpec((N_SEG, HID), lambda: (0, 0)),
            pl.BlockSpec((N_SEG, 1), lambda: (0, 0)),
            pl.BlockSpec((HID, HID), lambda: (0, 0)),
            pl.BlockSpec((HID, HID), lambda: (0, 0)),
            pl.BlockSpec((1, HID), lambda: (0, 0)),
        ],
        out_specs=pl.BlockSpec((N_SEG, HID), lambda: (0, 0)),
        out_shape=jax.ShapeDtypeStruct((N_SEG, HID), jnp.float32),
    )(s0, m0, cnt_col, Wpost[:HID], Wpost[HID:],
      bpost.reshape(1, HID))


def kernel(node_embeddings, batch, var_property_probs, node_types,
           Wp, bp, W1, b1, W2, b2, Wpost, bpost):
    del node_types
    hw0, cnt_col, starts = _run_dense(
        node_embeddings, var_property_probs, batch, Wp, bp, W1, b1, W2, b2)
    s0, m0 = _run_sc_reduce(hw0, starts, 0, N_TOTAL)
    return _run_combine(s0, m0, cnt_col, Wpost, bpost)


# speedup vs baseline: 1.2784x; 1.0849x over previous
import functools

import jax
import jax.numpy as jnp
from jax import lax
from jax.experimental import pallas as pl
from jax.experimental.pallas import tpu as pltpu
from jax.experimental.pallas import tpu_sc as plsc

N_TOTAL = 320000
N_SEG = 512
HID = 128
STARTS_LEN = 640

_SLICE_ROWS = N_TOTAL
_DENSE_R = 8000
_CHUNK = 896
_NW = 32
_SEG_PER_W = N_SEG // _NW

def _weight_scale(p_ref, w1_ref, b1_ref, w2_ref, b2_ref):
    hid = jnp.maximum(
        jnp.dot(p_ref[...], w1_ref[...], preferred_element_type=jnp.float32)
        + b1_ref[...], 0.0)
    z = jnp.sum(hid * w2_ref[...], axis=1, keepdims=True) + b2_ref[0, 0]
    return 1.0 / (1.0 + jnp.exp(-z))


def _dense_hist_kernel(x_ref, p_ref, b3_ref, wp_ref, bp_ref, w1_ref, b1_ref,
                       w2_ref, b2_ref, out_ref, counts_ref, starts_ref):
    t = pl.program_id(0)
    nt = pl.num_programs(0)

    @pl.when(t == 0)
    def _init():
        counts_ref[...] = jnp.zeros_like(counts_ref)

    b = b3_ref[0, 0, :]
    bmin = jnp.min(b)
    bmax = jnp.max(b)
    for w in range(N_SEG // 64):
        lo = w * 64

        @pl.when((bmin < lo + 64) & (bmax >= lo))
        def _win(lo=lo):
            ids = lo + lax.broadcasted_iota(jnp.int32, (1, 64), 1)
            oh = (b[:, None] == ids).astype(jnp.float32)
            counts_ref[:, lo:lo + 64] += jnp.sum(oh, axis=0)[None, :]

    h = jnp.dot(x_ref[...], wp_ref[...],
                preferred_element_type=jnp.float32) + bp_ref[...]
    w = _weight_scale(p_ref, w1_ref, b1_ref, w2_ref, b2_ref)
    out_ref[...] = h * w

    @pl.when(t == nt - 1)
    def _finalize():
        cnt = counts_ref[...]
        row = lax.broadcasted_iota(jnp.int32, (N_SEG, STARTS_LEN), 0)
        col = lax.broadcasted_iota(jnp.int32, (N_SEG, STARTS_LEN), 1)
        tri = (row < col).astype(jnp.float32)
        st = jnp.dot(cnt, tri, preferred_element_type=jnp.float32,
                     precision=lax.Precision.HIGHEST)
        starts_ref[...] = st.astype(jnp.int32)


def _run_dense(x, probs, batch, Wp, bp, W1, b1, W2, b2):
    nt = _SLICE_ROWS // _DENSE_R
    rb = N_TOTAL // nt
    batch3 = batch.reshape(nt, 1, rb)
    w1p = jnp.zeros((8, HID), jnp.float32).at[:, :32].set(W1)
    b1p = jnp.zeros((1, HID), jnp.float32).at[0, :32].set(b1)
    w2p = jnp.zeros((1, HID), jnp.float32).at[0, :32].set(W2[:, 0])
    b2p = jnp.full((1, HID), b2[0], jnp.float32)
    wpp = Wp
    bpp = bp.reshape(1, HID)

    wspecs = [
        pl.BlockSpec((HID, HID), lambda i: (0, 0)),
        pl.BlockSpec((1, HID), lambda i: (0, 0)),
        pl.BlockSpec((8, HID), lambda i: (0, 0)),
        pl.BlockSpec((1, HID), lambda i: (0, 0)),
        pl.BlockSpec((1, HID), lambda i: (0, 0)),
        pl.BlockSpec((1, HID), lambda i: (0, 0)),
    ]
    wargs = (wpp, bpp, w1p, b1p, w2p, b2p)

    hw0, counts, starts = pl.pallas_call(
        _dense_hist_kernel,
        grid=(nt,),
        in_specs=[
            pl.BlockSpec((_DENSE_R, HID), lambda i: (i, 0)),
            pl.BlockSpec((_DENSE_R, 8), lambda i: (i, 0)),
            pl.BlockSpec((1, 1, rb), lambda i: (i, 0, 0)),
        ] + wspecs,
        out_specs=[pl.BlockSpec((_DENSE_R, HID), lambda i: (i, 0)),
                   pl.BlockSpec((1, N_SEG), lambda i: (0, 0)),
                   pl.BlockSpec((1, STARTS_LEN), lambda i: (0, 0))],
        out_shape=[jax.ShapeDtypeStruct((_SLICE_ROWS, HID), jnp.float32),
                   jax.ShapeDtypeStruct((1, N_SEG), jnp.float32),
                   jax.ShapeDtypeStruct((1, STARTS_LEN), jnp.int32)],
    )(x, probs, batch3, *wargs)

    return hw0, counts.reshape(N_SEG, 1), starts.reshape(STARTS_LEN)


def _sc_reduce_body(hw_hbm, starts_hbm, sum_hbm, max_hbm, buf_v,
                    st_v, sum_v, max_v, sem, *, lo_s, hi_s):
    c = lax.axis_index("c")
    s = lax.axis_index("s")
    wid = s * 2 + c
    seg0 = wid * _SEG_PER_W
    n_s = hi_s - lo_s

    pltpu.sync_copy(starts_hbm.at[pl.ds(seg0, 24)], st_v)

    zero = jnp.zeros((16,), jnp.float32)
    ninf = jnp.full((16,), -jnp.inf, jnp.float32)
    for k in range(_SEG_PER_W):
        for cc in range(8):
            sum_v[pl.ds(k * HID + cc * 16, 16)] = zero
            max_v[pl.ds(k * HID + cc * 16, 16)] = ninf

    sa = st_v[pl.ds(0, 16)]
    sb = st_v[pl.ds(8, 16)]

    def stv(k):
        g = sa[k] if k < 16 else sb[k - 8]
        return jnp.clip(g, lo_s, hi_s) - lo_s

    r0 = stv(0)
    r1 = stv(_SEG_PER_W)
    nch = (r1 - r0 + _CHUNK - 1) // _CHUNK

    def process(buf, rcc, off):
        for k in range(_SEG_PER_W):
            lo = jnp.clip(stv(k) - rcc, off, _CHUNK)
            hi = jnp.clip(stv(k + 1) - rcc, off, _CHUNK)

            @pl.when(hi > lo)
            def _seg(k=k, lo=lo, hi=hi):
                accs = tuple(sum_v[pl.ds(k * HID + cc * 16, 16)] for cc in range(8))
                accm = tuple(max_v[pl.ds(k * HID + cc * 16, 16)] for cc in range(8))

                def row_body(j, acc):
                    new_s = [None] * 8
                    new_m = [None] * 8
                    for cc in range(8):
                        v = buf[pl.ds(j * HID + cc * 16, 16)]
                        new_s[cc] = acc[cc] + v
                        new_m[cc] = jnp.maximum(acc[8 + cc], v)
                    return tuple(new_s) + tuple(new_m)

                res = lax.fori_loop(lo, hi, row_body, accs + accm)
                for cc in range(8):
                    sum_v[pl.ds(k * HID + cc * 16, 16)] = res[cc]
                    max_v[pl.ds(k * HID + cc * 16, 16)] = res[8 + cc]

    def chunk_body(ci, carry):
        rc = r0 + ci * _CHUNK
        rcc = jnp.minimum(rc, n_s - _CHUNK)
        pltpu.async_copy(hw_hbm.at[pl.ds(rcc * HID, _CHUNK * HID)],
                         buf_v, sem).wait()
        process(buf_v, rcc, rc - rcc)
        return carry

    lax.fori_loop(0, nch, chunk_body, 0)

    pltpu.sync_copy(sum_v, sum_hbm.at[pl.ds(seg0 * HID, _SEG_PER_W * HID)])
    pltpu.sync_copy(max_v, max_hbm.at[pl.ds(seg0 * HID, _SEG_PER_W * HID)])


def _run_sc_reduce(hw, starts, lo_s, hi_s):
    mesh = plsc.VectorSubcoreMesh(core_axis_name="c", subcore_axis_name="s")
    body = functools.partial(_sc_reduce_body, lo_s=lo_s, hi_s=hi_s)
    kern = functools.partial(
        pl.kernel,
        mesh=mesh,
        out_type=[jax.ShapeDtypeStruct((N_SEG * HID,), jnp.float32),
                  jax.ShapeDtypeStruct((N_SEG * HID,), jnp.float32)],
        scratch_types=[
            pltpu.VMEM((_CHUNK * HID,), jnp.float32),
            pltpu.VMEM((24,), jnp.int32),
            pltpu.VMEM((_SEG_PER_W * HID,), jnp.float32),
            pltpu.VMEM((_SEG_PER_W * HID,), jnp.float32),
            pltpu.SemaphoreType.DMA,
        ],
    )(body)
    sum_f, max_f = kern(hw.reshape(_SLICE_ROWS * HID), starts)
    return sum_f.reshape(N_SEG, HID), max_f.reshape(N_SEG, HID)


def _combine_kernel(s0_ref, m0_ref, cnt_ref, wt_ref, wb_ref,
                    bp_ref, out_ref):
    r = 1.0 / jnp.maximum(cnt_ref[...], 1.0)
    mean = s0_ref[...] * r
    mx = m0_ref[...]
    out_ref[...] = (
        jnp.dot(mean, wt_ref[...], preferred_element_type=jnp.float32)
        + jnp.dot(mx, wb_ref[...], preferred_element_type=jnp.float32)
        + bp_ref[...])


def _run_combine(s0, m0, cnt_col, Wpost, bpost):
    return pl.pallas_call(
        _combine_kernel,
        in_specs=[
            pl.BlockSpec((N_SEG, HID), lambda: (0, 0)),
            pl.BlockSpec((N_SEG, HID), lambda: (0, 0)),
            pl.BlockSpec((N_SEG, 1), lambda: (0, 0)),
            pl.BlockSpec((HID, HID), lambda: (0, 0)),
            pl.BlockSpec((HID, HID), lambda: (0, 0)),
            pl.BlockSpec((1, HID), lambda: (0, 0)),
        ],
        out_specs=pl.BlockSpec((N_SEG, HID), lambda: (0, 0)),
        out_shape=jax.ShapeDtypeStruct((N_SEG, HID), jnp.float32),
    )(s0, m0, cnt_col, Wpost[:HID], Wpost[HID:],
      bpost.reshape(1, HID))


def kernel(node_embeddings, batch, var_property_probs, node_types,
           Wp, bp, W1, b1, W2, b2, Wpost, bpost):
    del node_types
    hw0, cnt_col, starts = _run_dense(
        node_embeddings, var_property_probs, batch, Wp, bp, W1, b1, W2, b2)
    s0, m0 = _run_sc_reduce(hw0, starts, 0, N_TOTAL)
    return _run_combine(s0, m0, cnt_col, Wpost, bpost)
